# baseline (speedup 1.0000x reference)
import jax
import jax.numpy as jnp
from jax import lax
from jax.experimental import pallas as pl
from jax.experimental.pallas import tpu as pltpu
from jax.experimental.pallas import tpu_sc as plsc

NUM_CORES = 2
NUM_SUBCORES = 16
NUM_WORKERS = NUM_CORES * NUM_SUBCORES
LANES = 16
BATCH = 16384
EMB = 64
NPAIR = 500000
BPW = BATCH // NUM_WORKERS
CHUNK = 128
NCHUNK = BPW // CHUNK
HALF = BPW // 2
HCHUNK = HALF // CHUNK

_GATHER_DNUMS = lax.GatherDimensionNumbers(
    offset_dims=(), collapsed_slice_dims=(0,), start_index_map=(0,))


def _shuffle(x, perm):
    return lax.gather(x, perm[:, None], dimension_numbers=_GATHER_DNUMS,
                      slice_sizes=(1,),
                      mode=lax.GatherScatterMode.PROMISE_IN_BOUNDS)


def _worker_base():
    wid = lax.axis_index("s") * NUM_CORES + lax.axis_index("c")
    return wid * BPW


def _gather_half(pair_hbm, idx_v, h, blk, rows_w, sem):
    for j in range(HCHUNK):
        for k in range(CHUNK // LANES):
            off = h * HALF + j * CHUNK + k * LANES
            blk[j, pl.ds(k * LANES, LANES)] = idx_v[pl.ds(off, LANES)] >> 1
    for j in range(HCHUNK):
        pltpu.async_copy(pair_hbm.at[blk.at[j]],
                         rows_w.at[pl.ds(j * CHUNK, CHUNK)], sem)
    for j in range(HCHUNK):
        pltpu.make_async_copy(pair_hbm.at[blk.at[j]],
                              rows_w.at[pl.ds(j * CHUNK, CHUNK)], sem).wait()


def _gather_user_body(u_hbm, ue_hbm, rows_out_hbm, u_idx, ublk, ue_w, sel_v, sem):
    base = _worker_base()
    for j in range(NCHUNK):
        pltpu.sync_copy(u_hbm.at[pl.ds(base + j * CHUNK, CHUNK)],
                        u_idx.at[pl.ds(j * CHUNK, CHUNK)])

    for h in range(2):
        _gather_half(ue_hbm, u_idx, h, ublk, ue_w, sem)

        def group(g, carry, h=h):
            gbase = pl.multiple_of(g * LANES, LANES)
            uh = (u_idx[pl.ds(h * HALF + gbase, LANES)] & 1).astype(jnp.float32)
            for r in range(LANES):
                slot = gbase + r
                hu = _shuffle(uh, jnp.full((LANES,), r, jnp.int32))
                for c in range(EMB // LANES):
                    lo = ue_w[slot, pl.ds(c * LANES, LANES)]
                    hi = ue_w[slot, pl.ds(EMB + c * LANES, LANES)]
                    sel_v[slot, pl.ds(c * LANES, LANES)] = lo + (hi - lo) * hu
            return carry

        lax.fori_loop(0, HALF // LANES, group, 0)
        pltpu.sync_copy(sel_v, rows_out_hbm.at[pl.ds(base + h * HALF, HALF)])


def _gather_dot_body(v_hbm, ve_hbm, ue_rows_hbm, out_hbm,
                     v_idx, vblk, ve_w, ue_v, out_v, sem):
    base = _worker_base()
    for j in range(NCHUNK):
        pltpu.sync_copy(v_hbm.at[pl.ds(base + j * CHUNK, CHUNK)],
                        v_idx.at[pl.ds(j * CHUNK, CHUNK)])

    lanes = lax.iota(jnp.int32, LANES)
    lanes_f = lanes.astype(jnp.float32)
    perms = [lanes ^ (1 << t) for t in range(4)]
    one = jnp.ones((LANES,), jnp.float32)
    onehots = [jnp.maximum(one - jnp.abs(lanes_f - float(r)), 0.0)
               for r in range(LANES)]

    for h in range(2):
        _gather_half(ve_hbm, v_idx, h, vblk, ve_w, sem)
        pltpu.sync_copy(ue_rows_hbm.at[pl.ds(base + h * HALF, HALF)], ue_v)

        def group(g, carry, h=h):
            gbase = pl.multiple_of(g * LANES, LANES)
            vh = (v_idx[pl.ds(h * HALF + gbase, LANES)] & 1).astype(jnp.float32)
            sums = jnp.zeros((LANES,), jnp.float32)
            for r in range(LANES):
                slot = gbase + r
                hv = _shuffle(vh, jnp.full((LANES,), r, jnp.int32))
                acc = jnp.zeros((LANES,), jnp.float32)
                for c in range(EMB // LANES):
                    vlo = ve_w[slot, pl.ds(c * LANES, LANES)]
                    vhi = ve_w[slot, pl.ds(EMB + c * LANES, LANES)]
                    ve = vlo + (vhi - vlo) * hv
                    ue = ue_v[slot, pl.ds(c * LANES, LANES)]
                    acc = acc + ue * ve
                for t in range(4):
                    acc = acc + _shuffle(acc, perms[t])
                sums = sums + acc * onehots[r]
            out_v[pl.ds(h * HALF + gbase, LANES)] = sums
            return carry

        lax.fori_loop(0, HALF // LANES, group, 0)

    pltpu.sync_copy(out_v, out_hbm.at[pl.ds(base, BPW)])


@jax.jit
def kernel(u, v, user_emb, item_emb):
    mesh = plsc.VectorSubcoreMesh(core_axis_name="c", subcore_axis_name="s",
                                  num_cores=NUM_CORES, num_subcores=NUM_SUBCORES)
    gather_user = pl.kernel(
        _gather_user_body,
        out_type=jax.ShapeDtypeStruct((BATCH, EMB), jnp.float32),
        mesh=mesh,
        scratch_types=[
            pltpu.VMEM((BPW,), jnp.int32),
            pltpu.VMEM((HCHUNK, CHUNK), jnp.int32),
            pltpu.VMEM((HALF, 2 * EMB), jnp.float32),
            pltpu.VMEM((HALF, EMB), jnp.float32),
            pltpu.SemaphoreType.DMA,
        ],
        compiler_params=pltpu.CompilerParams(use_tc_tiling_on_sc=True),
    )
    gather_dot = pl.kernel(
        _gather_dot_body,
        out_type=jax.ShapeDtypeStruct((BATCH,), jnp.float32),
        mesh=mesh,
        scratch_types=[
            pltpu.VMEM((BPW,), jnp.int32),
            pltpu.VMEM((HCHUNK, CHUNK), jnp.int32),
            pltpu.VMEM((HALF, 2 * EMB), jnp.float32),
            pltpu.VMEM((HALF, EMB), jnp.float32),
            pltpu.VMEM((BPW,), jnp.float32),
            pltpu.SemaphoreType.DMA,
        ],
        compiler_params=pltpu.CompilerParams(use_tc_tiling_on_sc=True),
    )
    ue_pair = user_emb.reshape(NPAIR, 2 * EMB)
    ve_pair = item_emb.reshape(NPAIR, 2 * EMB)
    ue_rows = gather_user(u, ue_pair)
    return gather_dot(v, ve_pair, ue_rows)
